# baseline (speedup 1.0000x reference)
import jax
import jax.numpy as jnp
from jax import lax
from jax.experimental import pallas as pl
from jax.experimental.pallas import tpu as pltpu
from jax.experimental.pallas import tpu_sc as plsc

L = 16
N_CELLS = 4096
N_GENES = 5000
NB = N_CELLS * N_GENES
N_BUCKETS = 20
BUCKET = NB // N_BUCKETS
HSZ = BUCKET + 128
CH = 4096
WCH = 4000
TSLICE = BUCKET // L
N_HID = 10
ROWS = 1024
SBUF = (ROWS + 1) * L
PARK = ROWS * L


def _sc_body(idx_hbm, w1_hbm, b1_hbm, w2_hbm, b2_hbm, out_hbm,
             ibuf, sbuf, ones, cbuf, obuf, zbuf, w1v, b1v, w2v, b2v, hist):
  c = lax.axis_index("c")
  s = lax.axis_index("s")
  n = idx_hbm.shape[0]
  shard = n // L
  nchunks = shard // CH

  def _fill(buf, nv, val, dtype):
    def body(i, x):
      buf[pl.ds(i * L, L)] = jnp.full((L,), val, dtype)
      return x
    lax.fori_loop(0, nv // L, body, 0)

  _fill(ones, SBUF, 1.0, jnp.float32)
  _fill(zbuf, WCH, 0.0, jnp.float32)
  pltpu.sync_copy(w1_hbm, w1v)
  pltpu.sync_copy(b1_hbm, b1v)
  pltpu.sync_copy(w2_hbm, w2v)
  pltpu.sync_copy(b2_hbm, b2v)

  def zslice(j, x):
    pltpu.sync_copy(zbuf, hist.at[pl.ds(s * TSLICE + j * WCH, WCH)])
    return x
  lax.fori_loop(0, TSLICE // WCH, zslice, 0)

  @pl.when(s == 0)
  def _():
    pltpu.sync_copy(zbuf.at[pl.ds(0, 128)], hist.at[pl.ds(BUCKET, 128)])

  plsc.subcore_barrier()

  dummy_vec = (jnp.full((L,), BUCKET, jnp.int32)
               + lax.iota(jnp.int32, L) * jnp.int32(8))
  lane = lax.iota(jnp.int32, L)
  zero_vec = jnp.zeros((L,), jnp.int32)
  one_vec = jnp.ones((L,), jnp.int32)

  def redummy():
    def body(t, x):
      sbuf[pl.ds(t * L, L)] = dummy_vec
      return x

    lax.fori_loop(0, ROWS + 1, body, 0)

  redummy()

  def flush():
    pltpu.sync_copy(ones, hist.at[sbuf], add=True)
    redummy()

  def process_chunk(bstatic, f, base):
    def vbody(g, f):
      v = ibuf[bstatic, pl.ds(g * L, L)]
      d = v - base
      m = (d >= jnp.int32(0)) & (d < jnp.int32(BUCKET))
      ring = (f & jnp.int32(ROWS - 1)) * jnp.int32(L) + lane
      pos = jnp.where(m, ring, jnp.int32(PARK) + lane)
      val = jnp.where(m, d, dummy_vec)
      plsc.store_scatter(sbuf, [pos], val)
      return f + jnp.where(m, one_vec, zero_vec)

    return lax.fori_loop(0, CH // L, vbody, f)

  def pass_body(p, x):
    base = (c * jnp.int32(N_BUCKETS // 2) + p) * jnp.int32(BUCKET)

    def src(j):
      return idx_hbm.at[pl.ds(s * shard + j * CH, CH)]

    def pair(jj, f):
      j0 = jj * 2
      pltpu.sync_copy(src(j0), ibuf.at[0])
      f = process_chunk(0, f, base)
      pltpu.sync_copy(src(j0 + 1), ibuf.at[1])
      f = process_chunk(1, f, base)
      sentv = sbuf[pl.ds(0, L)]
      smin = sentv[0]
      for k in range(1, L):
        smin = jnp.minimum(smin, sentv[k])

      @pl.when(smin < jnp.int32(BUCKET))
      def _():
        flush()

      return f

    lax.fori_loop(0, nchunks // 2, pair, zero_vec)
    flush()
    plsc.subcore_barrier()

    w1_vec = w1v[pl.ds(0, L)]
    b1_vec = b1v[pl.ds(0, L)]
    w2_vec = w2v[pl.ds(0, L)]
    b2_vec = b2v[pl.ds(0, L)]
    w1s = [w1_vec[k] for k in range(N_HID)]
    b1s = [b1_vec[k] for k in range(N_HID)]
    w2s = [w2_vec[k] for k in range(N_HID)]
    b2s = b2_vec[0]

    def wbody(j, y):
      off = s * TSLICE + j * WCH
      pltpu.sync_copy(hist.at[pl.ds(off, WCH)], cbuf)
      pltpu.sync_copy(zbuf, hist.at[pl.ds(off, WCH)])

      def mbody(g, z):
        cv = cbuf[pl.ds(g * L, L)]
        acc = jnp.full((L,), 0.0, jnp.float32) + b2s
        for k in range(N_HID):
          h = jnp.maximum(cv * w1s[k] + b1s[k], 0.0)
          acc = acc + h * w2s[k]
        obuf[pl.ds(g * L, L)] = acc
        return z

      lax.fori_loop(0, WCH // L, mbody, 0, unroll=2)
      pltpu.sync_copy(obuf, out_hbm.at[pl.ds(base + off, WCH)])
      return y

    lax.fori_loop(0, TSLICE // WCH, wbody, 0)
    plsc.subcore_barrier()
    return x

  lax.fori_loop(0, N_BUCKETS // 2, pass_body, 0)


@jax.jit
def _pooler(idx, w1p, b1p, w2p, b2p):
  mesh = plsc.VectorSubcoreMesh(core_axis_name="c", subcore_axis_name="s")
  f = pl.kernel(
      _sc_body,
      out_type=jax.ShapeDtypeStruct((NB,), jnp.float32),
      mesh=mesh,
      compiler_params=pltpu.CompilerParams(needs_layout_passes=False),
      scratch_types=[
          pltpu.VMEM((2, CH), jnp.int32),
          pltpu.VMEM((SBUF,), jnp.int32),
          pltpu.VMEM((SBUF,), jnp.float32),
          pltpu.VMEM((WCH,), jnp.float32),
          pltpu.VMEM((WCH,), jnp.float32),
          pltpu.VMEM((WCH,), jnp.float32),
          pltpu.VMEM((L,), jnp.float32),
          pltpu.VMEM((L,), jnp.float32),
          pltpu.VMEM((L,), jnp.float32),
          pltpu.VMEM((L,), jnp.float32),
          pltpu.VMEM_SHARED((HSZ,), jnp.float32),
      ],
  )
  return f(idx, w1p, b1p, w2p, b2p)


def kernel(cellxgene_ix, weights, n_cells, n_genes, W1, b1, W2, b2):
  del weights, n_cells, n_genes
  idx = cellxgene_ix.astype(jnp.int32)
  assert idx.shape[0] % (L * CH * 2) == 0

  def pad16(a):
    a = a.reshape(-1).astype(jnp.float32)
    return jnp.pad(a, (0, L - a.shape[0]))

  out = _pooler(idx, pad16(W1), pad16(b1), pad16(W2), pad16(b2))
  return out.reshape(N_CELLS, N_GENES)


# speedup vs baseline: 1.1663x; 1.1663x over previous
import jax
import jax.numpy as jnp
from jax import lax
from jax.experimental import pallas as pl
from jax.experimental.pallas import tpu as pltpu
from jax.experimental.pallas import tpu_sc as plsc

L = 16
N_CELLS = 4096
N_GENES = 5000
NB = N_CELLS * N_GENES
N_BUCKETS = 20
BUCKET = NB // N_BUCKETS
HSZ = BUCKET + 128
CH = 4096
WCH = 4000
TSLICE = BUCKET // L
N_HID = 10
ROWS = 1024
SBUF = (ROWS + 1) * L
PARK = ROWS * L


def _sc_body(idx_hbm, w1_hbm, b1_hbm, w2_hbm, b2_hbm, out_hbm,
             ibuf, sbuf, ones, cbuf, obuf, zbuf, w1v, b1v, w2v, b2v, hist,
             sem0, sem1):
  c = lax.axis_index("c")
  s = lax.axis_index("s")
  n = idx_hbm.shape[0]
  shard = n // L
  nchunks = shard // CH

  def _fill(buf, nv, val, dtype):
    def body(i, x):
      buf[pl.ds(i * L, L)] = jnp.full((L,), val, dtype)
      return x
    lax.fori_loop(0, nv // L, body, 0)

  _fill(ones, SBUF, 1.0, jnp.float32)
  _fill(zbuf, WCH, 0.0, jnp.float32)
  pltpu.sync_copy(w1_hbm, w1v)
  pltpu.sync_copy(b1_hbm, b1v)
  pltpu.sync_copy(w2_hbm, w2v)
  pltpu.sync_copy(b2_hbm, b2v)

  def zslice(j, x):
    pltpu.sync_copy(zbuf, hist.at[pl.ds(s * TSLICE + j * WCH, WCH)])
    return x
  lax.fori_loop(0, TSLICE // WCH, zslice, 0)

  @pl.when(s == 0)
  def _():
    pltpu.sync_copy(zbuf.at[pl.ds(0, 128)], hist.at[pl.ds(BUCKET, 128)])

  plsc.subcore_barrier()

  dummy_vec = (jnp.full((L,), BUCKET, jnp.int32)
               + lax.iota(jnp.int32, L) * jnp.int32(8))
  lane = lax.iota(jnp.int32, L)
  zero_vec = jnp.zeros((L,), jnp.int32)
  one_vec = jnp.ones((L,), jnp.int32)

  def redummy():
    def body(t, x):
      sbuf[pl.ds(t * L, L)] = dummy_vec
      return x

    lax.fori_loop(0, ROWS + 1, body, 0)

  redummy()

  def flush():
    pltpu.sync_copy(ones, hist.at[sbuf], add=True)
    redummy()

  def process_chunk(bstatic, f, base):
    def vbody(g, f):
      v = ibuf[bstatic, pl.ds(g * L, L)]
      d = v - base
      m = (d >= jnp.int32(0)) & (d < jnp.int32(BUCKET))
      ring = (f & jnp.int32(ROWS - 1)) * jnp.int32(L) + lane
      pos = jnp.where(m, ring, jnp.int32(PARK) + lane)
      val = jnp.where(m, d, dummy_vec)
      plsc.store_scatter(sbuf, [pos], val)
      return f + jnp.where(m, one_vec, zero_vec)

    return lax.fori_loop(0, CH // L, vbody, f, unroll=8)

  def pass_body(p, x):
    base = (c * jnp.int32(N_BUCKETS // 2) + p) * jnp.int32(BUCKET)

    def src(j):
      return idx_hbm.at[pl.ds(s * shard + j * CH, CH)]

    pltpu.async_copy(src(0), ibuf.at[0], sem0)

    def pair(jj, f):
      j0 = jj * 2
      pltpu.async_copy(src(j0 + 1), ibuf.at[1], sem1)
      pltpu.make_async_copy(src(0), ibuf.at[0], sem0).wait()
      f = process_chunk(0, f, base)

      @pl.when(jj + 1 < nchunks // 2)
      def _():
        pltpu.async_copy(src(j0 + 2), ibuf.at[0], sem0)

      pltpu.make_async_copy(src(0), ibuf.at[1], sem1).wait()
      f = process_chunk(1, f, base)
      sentv = sbuf[pl.ds(0, L)]
      smin = sentv[0]
      for k in range(1, L):
        smin = jnp.minimum(smin, sentv[k])

      @pl.when(smin < jnp.int32(BUCKET))
      def _():
        flush()

      return f

    lax.fori_loop(0, nchunks // 2, pair, zero_vec)
    flush()
    plsc.subcore_barrier()

    w1_vec = w1v[pl.ds(0, L)]
    b1_vec = b1v[pl.ds(0, L)]
    w2_vec = w2v[pl.ds(0, L)]
    b2_vec = b2v[pl.ds(0, L)]
    w1s = [w1_vec[k] for k in range(N_HID)]
    b1s = [b1_vec[k] for k in range(N_HID)]
    w2s = [w2_vec[k] for k in range(N_HID)]
    b2s = b2_vec[0]

    def wbody(j, y):
      off = s * TSLICE + j * WCH
      pltpu.sync_copy(hist.at[pl.ds(off, WCH)], cbuf)
      pltpu.sync_copy(zbuf, hist.at[pl.ds(off, WCH)])

      def mbody(g, z):
        cv = cbuf[pl.ds(g * L, L)]
        acc = jnp.full((L,), 0.0, jnp.float32) + b2s
        for k in range(N_HID):
          h = jnp.maximum(cv * w1s[k] + b1s[k], 0.0)
          acc = acc + h * w2s[k]
        obuf[pl.ds(g * L, L)] = acc
        return z

      lax.fori_loop(0, WCH // L, mbody, 0, unroll=2)
      pltpu.sync_copy(obuf, out_hbm.at[pl.ds(base + off, WCH)])
      return y

    lax.fori_loop(0, TSLICE // WCH, wbody, 0)
    plsc.subcore_barrier()
    return x

  lax.fori_loop(0, N_BUCKETS // 2, pass_body, 0)


@jax.jit
def _pooler(idx, w1p, b1p, w2p, b2p):
  mesh = plsc.VectorSubcoreMesh(core_axis_name="c", subcore_axis_name="s")
  f = pl.kernel(
      _sc_body,
      out_type=jax.ShapeDtypeStruct((NB,), jnp.float32),
      mesh=mesh,
      compiler_params=pltpu.CompilerParams(needs_layout_passes=False),
      scratch_types=[
          pltpu.VMEM((2, CH), jnp.int32),
          pltpu.VMEM((SBUF,), jnp.int32),
          pltpu.VMEM((SBUF,), jnp.float32),
          pltpu.VMEM((WCH,), jnp.float32),
          pltpu.VMEM((WCH,), jnp.float32),
          pltpu.VMEM((WCH,), jnp.float32),
          pltpu.VMEM((L,), jnp.float32),
          pltpu.VMEM((L,), jnp.float32),
          pltpu.VMEM((L,), jnp.float32),
          pltpu.VMEM((L,), jnp.float32),
          pltpu.VMEM_SHARED((HSZ,), jnp.float32),
          pltpu.SemaphoreType.DMA,
          pltpu.SemaphoreType.DMA,
      ],
  )
  return f(idx, w1p, b1p, w2p, b2p)


def kernel(cellxgene_ix, weights, n_cells, n_genes, W1, b1, W2, b2):
  del weights, n_cells, n_genes
  idx = cellxgene_ix.astype(jnp.int32)
  assert idx.shape[0] % (L * CH * 2) == 0

  def pad16(a):
    a = a.reshape(-1).astype(jnp.float32)
    return jnp.pad(a, (0, L - a.shape[0]))

  out = _pooler(idx, pad16(W1), pad16(b1), pad16(W2), pad16(b2))
  return out.reshape(N_CELLS, N_GENES)


# speedup vs baseline: 1.1664x; 1.0001x over previous
import jax
import jax.numpy as jnp
from jax import lax
from jax.experimental import pallas as pl
from jax.experimental.pallas import tpu as pltpu
from jax.experimental.pallas import tpu_sc as plsc

L = 16
N_CELLS = 4096
N_GENES = 5000
NB = N_CELLS * N_GENES
N_BUCKETS = 20
BUCKET = NB // N_BUCKETS
HSZ = BUCKET + 128
CH = 8192
WCH = 4000
TSLICE = BUCKET // L
N_HID = 10
ROWS = 1024
SBUF = (ROWS + 1) * L
PARK = ROWS * L


def _sc_body(idx_hbm, w1_hbm, b1_hbm, w2_hbm, b2_hbm, out_hbm,
             ibuf, sbuf, ones, cbuf, obuf, zbuf, w1v, b1v, w2v, b2v, hist,
             sem0, sem1):
  c = lax.axis_index("c")
  s = lax.axis_index("s")
  n = idx_hbm.shape[0]
  shard = n // L
  nchunks = shard // CH

  def _fill(buf, nv, val, dtype):
    def body(i, x):
      buf[pl.ds(i * L, L)] = jnp.full((L,), val, dtype)
      return x
    lax.fori_loop(0, nv // L, body, 0)

  _fill(ones, SBUF, 1.0, jnp.float32)
  _fill(zbuf, WCH, 0.0, jnp.float32)
  pltpu.sync_copy(w1_hbm, w1v)
  pltpu.sync_copy(b1_hbm, b1v)
  pltpu.sync_copy(w2_hbm, w2v)
  pltpu.sync_copy(b2_hbm, b2v)

  def zslice(j, x):
    pltpu.sync_copy(zbuf, hist.at[pl.ds(s * TSLICE + j * WCH, WCH)])
    return x
  lax.fori_loop(0, TSLICE // WCH, zslice, 0)

  @pl.when(s == 0)
  def _():
    pltpu.sync_copy(zbuf.at[pl.ds(0, 128)], hist.at[pl.ds(BUCKET, 128)])

  plsc.subcore_barrier()

  dummy_vec = (jnp.full((L,), BUCKET, jnp.int32)
               + lax.iota(jnp.int32, L) * jnp.int32(8))
  lane = lax.iota(jnp.int32, L)
  zero_vec = jnp.zeros((L,), jnp.int32)
  l_vec = jnp.full((L,), L, jnp.int32)

  def redummy():
    def body(t, x):
      sbuf[pl.ds(t * L, L)] = dummy_vec
      return x

    lax.fori_loop(0, ROWS + 1, body, 0)

  redummy()

  def flush():
    pltpu.sync_copy(ones, hist.at[sbuf], add=True)
    redummy()

  def process_chunk(bstatic, f, base):
    def vbody(g, f):
      v = ibuf[bstatic, pl.ds(g * L, L)]
      d = v - base
      m = (d >= jnp.int32(0)) & (d < jnp.int32(BUCKET))
      ring = (f & jnp.int32(PARK - L)) + lane
      pos = jnp.where(m, ring, jnp.int32(PARK) + lane)
      val = jnp.where(m, d, dummy_vec)
      plsc.store_scatter(sbuf, [pos], val)
      return f + jnp.where(m, l_vec, zero_vec)

    return lax.fori_loop(0, CH // L, vbody, f, unroll=8)

  def pass_body(p, x):
    base = (c * jnp.int32(N_BUCKETS // 2) + p) * jnp.int32(BUCKET)

    def src(j):
      return idx_hbm.at[pl.ds(s * shard + j * CH, CH)]

    def check_flush():
      sentv = sbuf[pl.ds(0, L)]
      smin = sentv[0]
      for k in range(1, L):
        smin = jnp.minimum(smin, sentv[k])

      @pl.when(smin < jnp.int32(BUCKET))
      def _():
        flush()

    pltpu.async_copy(src(0), ibuf.at[0], sem0)

    def pair(jj, f):
      j0 = jj * 2
      pltpu.async_copy(src(j0 + 1), ibuf.at[1], sem1)
      pltpu.make_async_copy(src(0), ibuf.at[0], sem0).wait()
      f = process_chunk(0, f, base)
      check_flush()

      @pl.when(jj + 1 < nchunks // 2)
      def _():
        pltpu.async_copy(src(j0 + 2), ibuf.at[0], sem0)

      pltpu.make_async_copy(src(0), ibuf.at[1], sem1).wait()
      f = process_chunk(1, f, base)
      check_flush()
      return f

    lax.fori_loop(0, nchunks // 2, pair, zero_vec)
    flush()
    plsc.subcore_barrier()

    w1_vec = w1v[pl.ds(0, L)]
    b1_vec = b1v[pl.ds(0, L)]
    w2_vec = w2v[pl.ds(0, L)]
    b2_vec = b2v[pl.ds(0, L)]
    w1s = [w1_vec[k] for k in range(N_HID)]
    b1s = [b1_vec[k] for k in range(N_HID)]
    w2s = [w2_vec[k] for k in range(N_HID)]
    b2s = b2_vec[0]

    def wbody(j, y):
      off = s * TSLICE + j * WCH
      pltpu.sync_copy(hist.at[pl.ds(off, WCH)], cbuf)
      pltpu.sync_copy(zbuf, hist.at[pl.ds(off, WCH)])

      def mbody(g, z):
        cv = cbuf[pl.ds(g * L, L)]
        acc = jnp.full((L,), 0.0, jnp.float32) + b2s
        for k in range(N_HID):
          h = jnp.maximum(cv * w1s[k] + b1s[k], 0.0)
          acc = acc + h * w2s[k]
        obuf[pl.ds(g * L, L)] = acc
        return z

      lax.fori_loop(0, WCH // L, mbody, 0, unroll=2)
      pltpu.sync_copy(obuf, out_hbm.at[pl.ds(base + off, WCH)])
      return y

    lax.fori_loop(0, TSLICE // WCH, wbody, 0)
    plsc.subcore_barrier()
    return x

  lax.fori_loop(0, N_BUCKETS // 2, pass_body, 0)


@jax.jit
def _pooler(idx, w1p, b1p, w2p, b2p):
  mesh = plsc.VectorSubcoreMesh(core_axis_name="c", subcore_axis_name="s")
  f = pl.kernel(
      _sc_body,
      out_type=jax.ShapeDtypeStruct((NB,), jnp.float32),
      mesh=mesh,
      compiler_params=pltpu.CompilerParams(needs_layout_passes=False),
      scratch_types=[
          pltpu.VMEM((2, CH), jnp.int32),
          pltpu.VMEM((SBUF,), jnp.int32),
          pltpu.VMEM((SBUF,), jnp.float32),
          pltpu.VMEM((WCH,), jnp.float32),
          pltpu.VMEM((WCH,), jnp.float32),
          pltpu.VMEM((WCH,), jnp.float32),
          pltpu.VMEM((L,), jnp.float32),
          pltpu.VMEM((L,), jnp.float32),
          pltpu.VMEM((L,), jnp.float32),
          pltpu.VMEM((L,), jnp.float32),
          pltpu.VMEM_SHARED((HSZ,), jnp.float32),
          pltpu.SemaphoreType.DMA,
          pltpu.SemaphoreType.DMA,
      ],
  )
  return f(idx, w1p, b1p, w2p, b2p)


def kernel(cellxgene_ix, weights, n_cells, n_genes, W1, b1, W2, b2):
  del weights, n_cells, n_genes
  idx = cellxgene_ix.astype(jnp.int32)
  assert idx.shape[0] % (L * CH * 2) == 0

  def pad16(a):
    a = a.reshape(-1).astype(jnp.float32)
    return jnp.pad(a, (0, L - a.shape[0]))

  out = _pooler(idx, pad16(W1), pad16(b1), pad16(W2), pad16(b2))
  return out.reshape(N_CELLS, N_GENES)


# speedup vs baseline: 1.6613x; 1.4243x over previous
import jax
import jax.numpy as jnp
from jax import lax
from jax.experimental import pallas as pl
from jax.experimental.pallas import tpu as pltpu
from jax.experimental.pallas import tpu_sc as plsc

L = 16
N_CELLS = 4096
N_GENES = 5000
NB = N_CELLS * N_GENES
N_BUCKETS = 20
BUCKET = NB // N_BUCKETS
HSZ = BUCKET + 128
CH = 8192
WCH = 4000
TSLICE = BUCKET // L
N_HID = 10
ROWS = 1024
SBUF = (ROWS + 1) * L
PARK = ROWS * L


def _sc_body(idx_hbm, w1_hbm, b1_hbm, w2_hbm, b2_hbm, out_hbm,
             ibuf, sbuf, ones, cbuf, obuf, zbuf, w1v, b1v, w2v, b2v, hist,
             sem0, sem1):
  c = lax.axis_index("c")
  s = lax.axis_index("s")
  n = idx_hbm.shape[0]
  shard = n // L
  nchunks = shard // CH

  def _fill(buf, nv, val, dtype):
    def body(i, x):
      buf[pl.ds(i * L, L)] = jnp.full((L,), val, dtype)
      return x
    lax.fori_loop(0, nv // L, body, 0)

  _fill(ones, SBUF, 1.0, jnp.float32)
  _fill(zbuf, WCH, 0.0, jnp.float32)
  pltpu.sync_copy(w1_hbm, w1v)
  pltpu.sync_copy(b1_hbm, b1v)
  pltpu.sync_copy(w2_hbm, w2v)
  pltpu.sync_copy(b2_hbm, b2v)

  def zslice(j, x):
    pltpu.sync_copy(zbuf, hist.at[pl.ds(s * TSLICE + j * WCH, WCH)])
    return x
  lax.fori_loop(0, TSLICE // WCH, zslice, 0)

  @pl.when(s == 0)
  def _():
    pltpu.sync_copy(zbuf.at[pl.ds(0, 128)], hist.at[pl.ds(BUCKET, 128)])

  plsc.subcore_barrier()

  dummy_vec = (jnp.full((L,), BUCKET, jnp.int32)
               + lax.iota(jnp.int32, L) * jnp.int32(8))
  lane = lax.iota(jnp.int32, L)
  zero_vec = jnp.zeros((L,), jnp.int32)
  l_vec = jnp.full((L,), L, jnp.int32)

  def redummy():
    def body(t, x):
      sbuf[pl.ds(t * L, L)] = dummy_vec
      return x

    lax.fori_loop(0, ROWS + 1, body, 0)

  redummy()

  def flush():
    pltpu.sync_copy(ones, hist.at[sbuf], add=True)
    redummy()

  def process_chunk(bstatic, f, base):
    @plsc.parallel_loop(0, CH // L, carry=f, unroll=8)
    def vbody(g, f):
      v = ibuf[bstatic, pl.ds(g * L, L)]
      d = v - base
      m = (d >= jnp.int32(0)) & (d < jnp.int32(BUCKET))
      ring = (f & jnp.int32(PARK - L)) + lane
      pos = jnp.where(m, ring, jnp.int32(PARK) + lane)
      val = jnp.where(m, d, dummy_vec)
      plsc.store_scatter(sbuf, [pos], val)
      return f + jnp.where(m, l_vec, zero_vec)

    return vbody

  def pass_body(p, x):
    base = (c * jnp.int32(N_BUCKETS // 2) + p) * jnp.int32(BUCKET)

    def src(j):
      return idx_hbm.at[pl.ds(s * shard + j * CH, CH)]

    def check_flush():
      sentv = sbuf[pl.ds(0, L)]
      smin = sentv[0]
      for k in range(1, L):
        smin = jnp.minimum(smin, sentv[k])

      @pl.when(smin < jnp.int32(BUCKET))
      def _():
        flush()

    pltpu.async_copy(src(0), ibuf.at[0], sem0)

    def pair(jj, f):
      j0 = jj * 2
      pltpu.async_copy(src(j0 + 1), ibuf.at[1], sem1)
      pltpu.make_async_copy(src(0), ibuf.at[0], sem0).wait()
      f = process_chunk(0, f, base)
      check_flush()

      @pl.when(jj + 1 < nchunks // 2)
      def _():
        pltpu.async_copy(src(j0 + 2), ibuf.at[0], sem0)

      pltpu.make_async_copy(src(0), ibuf.at[1], sem1).wait()
      f = process_chunk(1, f, base)
      check_flush()
      return f

    lax.fori_loop(0, nchunks // 2, pair, zero_vec)
    flush()
    plsc.subcore_barrier()

    w1_vec = w1v[pl.ds(0, L)]
    b1_vec = b1v[pl.ds(0, L)]
    w2_vec = w2v[pl.ds(0, L)]
    b2_vec = b2v[pl.ds(0, L)]
    w1s = [w1_vec[k] for k in range(N_HID)]
    b1s = [b1_vec[k] for k in range(N_HID)]
    w2s = [w2_vec[k] for k in range(N_HID)]
    b2s = b2_vec[0]

    def wbody(j, y):
      off = s * TSLICE + j * WCH
      pltpu.sync_copy(hist.at[pl.ds(off, WCH)], cbuf)
      pltpu.sync_copy(zbuf, hist.at[pl.ds(off, WCH)])

      def mbody(g, z):
        cv = cbuf[pl.ds(g * L, L)]
        acc = jnp.full((L,), 0.0, jnp.float32) + b2s
        for k in range(N_HID):
          h = jnp.maximum(cv * w1s[k] + b1s[k], 0.0)
          acc = acc + h * w2s[k]
        obuf[pl.ds(g * L, L)] = acc
        return z

      lax.fori_loop(0, WCH // L, mbody, 0, unroll=2)
      pltpu.sync_copy(obuf, out_hbm.at[pl.ds(base + off, WCH)])
      return y

    lax.fori_loop(0, TSLICE // WCH, wbody, 0)
    plsc.subcore_barrier()
    return x

  lax.fori_loop(0, N_BUCKETS // 2, pass_body, 0)


@jax.jit
def _pooler(idx, w1p, b1p, w2p, b2p):
  mesh = plsc.VectorSubcoreMesh(core_axis_name="c", subcore_axis_name="s")
  f = pl.kernel(
      _sc_body,
      out_type=jax.ShapeDtypeStruct((NB,), jnp.float32),
      mesh=mesh,
      compiler_params=pltpu.CompilerParams(needs_layout_passes=False),
      scratch_types=[
          pltpu.VMEM((2, CH), jnp.int32),
          pltpu.VMEM((SBUF,), jnp.int32),
          pltpu.VMEM((SBUF,), jnp.float32),
          pltpu.VMEM((WCH,), jnp.float32),
          pltpu.VMEM((WCH,), jnp.float32),
          pltpu.VMEM((WCH,), jnp.float32),
          pltpu.VMEM((L,), jnp.float32),
          pltpu.VMEM((L,), jnp.float32),
          pltpu.VMEM((L,), jnp.float32),
          pltpu.VMEM((L,), jnp.float32),
          pltpu.VMEM_SHARED((HSZ,), jnp.float32),
          pltpu.SemaphoreType.DMA,
          pltpu.SemaphoreType.DMA,
      ],
  )
  return f(idx, w1p, b1p, w2p, b2p)


def kernel(cellxgene_ix, weights, n_cells, n_genes, W1, b1, W2, b2):
  del weights, n_cells, n_genes
  idx = cellxgene_ix.astype(jnp.int32)
  assert idx.shape[0] % (L * CH * 2) == 0

  def pad16(a):
    a = a.reshape(-1).astype(jnp.float32)
    return jnp.pad(a, (0, L - a.shape[0]))

  out = _pooler(idx, pad16(W1), pad16(b1), pad16(W2), pad16(b2))
  return out.reshape(N_CELLS, N_GENES)


# speedup vs baseline: 1.8900x; 1.1376x over previous
import jax
import jax.numpy as jnp
from jax import lax
from jax.experimental import pallas as pl
from jax.experimental.pallas import tpu as pltpu
from jax.experimental.pallas import tpu_sc as plsc

L = 16
N_CELLS = 4096
N_GENES = 5000
NB = N_CELLS * N_GENES
N_BUCKETS = 20
BUCKET = NB // N_BUCKETS
HSZ = BUCKET + 128
CH = 8192
WCH = 4000
TSLICE = BUCKET // L
N_HID = 10
ROWS = 1024
SBUF = (ROWS + 1) * L
PARK = ROWS * L


def _sc_body(idx_hbm, w1_hbm, b1_hbm, w2_hbm, b2_hbm, out_hbm,
             ibuf, sbuf, ones, cbuf, obuf, zbuf, w1v, b1v, w2v, b2v, hist,
             sem0, sem1):
  c = lax.axis_index("c")
  s = lax.axis_index("s")
  n = idx_hbm.shape[0]
  shard = n // L
  nchunks = shard // CH

  def _fill(buf, nv, val, dtype):
    @plsc.parallel_loop(0, nv // L, unroll=8)
    def body(i):
      buf[pl.ds(i * L, L)] = jnp.full((L,), val, dtype)

  _fill(ones, SBUF, 1.0, jnp.float32)
  _fill(zbuf, WCH, 0.0, jnp.float32)
  pltpu.sync_copy(w1_hbm, w1v)
  pltpu.sync_copy(b1_hbm, b1v)
  pltpu.sync_copy(w2_hbm, w2v)
  pltpu.sync_copy(b2_hbm, b2v)

  def zslice(j, x):
    pltpu.sync_copy(zbuf, hist.at[pl.ds(s * TSLICE + j * WCH, WCH)])
    return x
  lax.fori_loop(0, TSLICE // WCH, zslice, 0)

  @pl.when(s == 0)
  def _():
    pltpu.sync_copy(zbuf.at[pl.ds(0, 128)], hist.at[pl.ds(BUCKET, 128)])

  plsc.subcore_barrier()

  dummy_vec = (jnp.full((L,), BUCKET, jnp.int32)
               + lax.iota(jnp.int32, L) * jnp.int32(8))
  lane = lax.iota(jnp.int32, L)
  zero_vec = jnp.zeros((L,), jnp.int32)
  l_vec = jnp.full((L,), L, jnp.int32)

  def redummy():
    @plsc.parallel_loop(0, ROWS + 1, unroll=8)
    def body(t):
      sbuf[pl.ds(t * L, L)] = dummy_vec

  redummy()

  def flush():
    pltpu.sync_copy(ones, hist.at[sbuf], add=True)
    redummy()

  def process_chunk(bstatic, f, base):
    @plsc.parallel_loop(0, CH // L, carry=f, unroll=8)
    def vbody(g, f):
      v = ibuf[bstatic, pl.ds(g * L, L)]
      d = v - base
      m = (d >= jnp.int32(0)) & (d < jnp.int32(BUCKET))
      ring = (f & jnp.int32(PARK - L)) + lane
      pos = jnp.where(m, ring, jnp.int32(PARK) + lane)
      val = jnp.where(m, d, dummy_vec)
      plsc.store_scatter(sbuf, [pos], val)
      return f + jnp.where(m, l_vec, zero_vec)

    return vbody

  def pass_body(p, x):
    base = (c * jnp.int32(N_BUCKETS // 2) + p) * jnp.int32(BUCKET)

    def src(j):
      return idx_hbm.at[pl.ds(s * shard + j * CH, CH)]

    def check_flush():
      sentv = sbuf[pl.ds(0, L)]
      smin = sentv[0]
      for k in range(1, L):
        smin = jnp.minimum(smin, sentv[k])

      @pl.when(smin < jnp.int32(BUCKET))
      def _():
        flush()

    pltpu.async_copy(src(0), ibuf.at[0], sem0)

    def pair(jj, f):
      j0 = jj * 2
      pltpu.async_copy(src(j0 + 1), ibuf.at[1], sem1)
      pltpu.make_async_copy(src(0), ibuf.at[0], sem0).wait()
      f = process_chunk(0, f, base)
      check_flush()

      @pl.when(jj + 1 < nchunks // 2)
      def _():
        pltpu.async_copy(src(j0 + 2), ibuf.at[0], sem0)

      pltpu.make_async_copy(src(0), ibuf.at[1], sem1).wait()
      f = process_chunk(1, f, base)
      check_flush()
      return f

    lax.fori_loop(0, nchunks // 2, pair, zero_vec)
    flush()
    plsc.subcore_barrier()

    w1_vec = w1v[pl.ds(0, L)]
    b1_vec = b1v[pl.ds(0, L)]
    w2_vec = w2v[pl.ds(0, L)]
    b2_vec = b2v[pl.ds(0, L)]
    w1s = [w1_vec[k] for k in range(N_HID)]
    b1s = [b1_vec[k] for k in range(N_HID)]
    w2s = [w2_vec[k] for k in range(N_HID)]
    b2s = b2_vec[0]

    def wbody(j, y):
      off = s * TSLICE + j * WCH
      pltpu.sync_copy(hist.at[pl.ds(off, WCH)], cbuf)
      pltpu.sync_copy(zbuf, hist.at[pl.ds(off, WCH)])

      @plsc.parallel_loop(0, WCH // L, unroll=4)
      def mbody(g):
        cv = cbuf[pl.ds(g * L, L)]
        acc = jnp.full((L,), 0.0, jnp.float32) + b2s
        for k in range(N_HID):
          h = jnp.maximum(cv * w1s[k] + b1s[k], 0.0)
          acc = acc + h * w2s[k]
        obuf[pl.ds(g * L, L)] = acc
      pltpu.sync_copy(obuf, out_hbm.at[pl.ds(base + off, WCH)])
      return y

    lax.fori_loop(0, TSLICE // WCH, wbody, 0)
    plsc.subcore_barrier()
    return x

  lax.fori_loop(0, N_BUCKETS // 2, pass_body, 0)


@jax.jit
def _pooler(idx, w1p, b1p, w2p, b2p):
  mesh = plsc.VectorSubcoreMesh(core_axis_name="c", subcore_axis_name="s")
  f = pl.kernel(
      _sc_body,
      out_type=jax.ShapeDtypeStruct((NB,), jnp.float32),
      mesh=mesh,
      compiler_params=pltpu.CompilerParams(needs_layout_passes=False),
      scratch_types=[
          pltpu.VMEM((2, CH), jnp.int32),
          pltpu.VMEM((SBUF,), jnp.int32),
          pltpu.VMEM((SBUF,), jnp.float32),
          pltpu.VMEM((WCH,), jnp.float32),
          pltpu.VMEM((WCH,), jnp.float32),
          pltpu.VMEM((WCH,), jnp.float32),
          pltpu.VMEM((L,), jnp.float32),
          pltpu.VMEM((L,), jnp.float32),
          pltpu.VMEM((L,), jnp.float32),
          pltpu.VMEM((L,), jnp.float32),
          pltpu.VMEM_SHARED((HSZ,), jnp.float32),
          pltpu.SemaphoreType.DMA,
          pltpu.SemaphoreType.DMA,
      ],
  )
  return f(idx, w1p, b1p, w2p, b2p)


def kernel(cellxgene_ix, weights, n_cells, n_genes, W1, b1, W2, b2):
  del weights, n_cells, n_genes
  idx = cellxgene_ix.astype(jnp.int32)
  assert idx.shape[0] % (L * CH * 2) == 0

  def pad16(a):
    a = a.reshape(-1).astype(jnp.float32)
    return jnp.pad(a, (0, L - a.shape[0]))

  out = _pooler(idx, pad16(W1), pad16(b1), pad16(W2), pad16(b2))
  return out.reshape(N_CELLS, N_GENES)


# speedup vs baseline: 1.9657x; 1.0400x over previous
import jax
import jax.numpy as jnp
from jax import lax
from jax.experimental import pallas as pl
from jax.experimental.pallas import tpu as pltpu
from jax.experimental.pallas import tpu_sc as plsc

L = 16
N_CELLS = 4096
N_GENES = 5000
NB = N_CELLS * N_GENES
N_BUCKETS = 20
BUCKET = NB // N_BUCKETS
HSZ = BUCKET + 128
CH = 8192
WCH = 4000
TSLICE = BUCKET // L
N_HID = 10
ROWS = 1024
SBUF = (ROWS + 1) * L
PARK = ROWS * L


def _sc_body(idx_hbm, w1_hbm, b1_hbm, w2_hbm, b2_hbm, out_hbm,
             ibuf, sbuf, ones, cbuf, obuf, zbuf, w1v, b1v, w2v, b2v, hist,
             sem0, sem1):
  c = lax.axis_index("c")
  s = lax.axis_index("s")
  n = idx_hbm.shape[0]
  shard = n // L
  nchunks = shard // CH

  def _fill(buf, nv, val, dtype):
    @plsc.parallel_loop(0, nv // L, unroll=8)
    def body(i):
      buf[pl.ds(i * L, L)] = jnp.full((L,), val, dtype)

  _fill(ones, SBUF, 1.0, jnp.float32)
  _fill(zbuf, WCH, 0.0, jnp.float32)
  pltpu.sync_copy(w1_hbm, w1v)
  pltpu.sync_copy(b1_hbm, b1v)
  pltpu.sync_copy(w2_hbm, w2v)
  pltpu.sync_copy(b2_hbm, b2v)

  def zslice(j, x):
    pltpu.sync_copy(zbuf, hist.at[pl.ds(s * TSLICE + j * WCH, WCH)])
    return x
  lax.fori_loop(0, TSLICE // WCH, zslice, 0)

  @pl.when(s == 0)
  def _():
    pltpu.sync_copy(zbuf.at[pl.ds(0, 128)], hist.at[pl.ds(BUCKET, 128)])


  plsc.subcore_barrier()

  dummy_vec = (jnp.full((L,), BUCKET, jnp.int32)
               + lax.iota(jnp.int32, L) * jnp.int32(8))
  lane = lax.iota(jnp.int32, L)
  zero_vec = jnp.zeros((L,), jnp.int32)
  l_vec = jnp.full((L,), L, jnp.int32)

  def redummy():
    @plsc.parallel_loop(0, ROWS + 1, unroll=8)
    def body(t):
      sbuf[pl.ds(t * L, L)] = dummy_vec

  redummy()

  def flush():
    pltpu.sync_copy(ones, hist.at[sbuf], add=True)
    redummy()

  def process_chunk(bstatic, f, base):
    @plsc.parallel_loop(0, CH // L, carry=f, unroll=16)
    def vbody(g, f):
      v = ibuf[bstatic, pl.ds(g * L, L)]
      d = v - base
      m = (d >= jnp.int32(0)) & (d < jnp.int32(BUCKET))
      ring = (f & jnp.int32(PARK - L)) + lane
      pos = jnp.where(m, ring, jnp.int32(PARK) + lane)
      val = jnp.where(m, d, dummy_vec)
      plsc.store_scatter(sbuf, [pos], val)
      return f + jnp.where(m, l_vec, zero_vec)

    return vbody

  def pass_body(p, x):
    base = (c * jnp.int32(N_BUCKETS // 2) + p) * jnp.int32(BUCKET)

    def src(j):
      return idx_hbm.at[pl.ds(s * shard + j * CH, CH)]

    def check_flush():
      sentv = sbuf[pl.ds(0, L)]
      smin = sentv[0]
      for k in range(1, L):
        smin = jnp.minimum(smin, sentv[k])

      @pl.when(smin < jnp.int32(BUCKET))
      def _():
        flush()

    pltpu.async_copy(src(0), ibuf.at[0], sem0)

    def pair(jj, f):
      j0 = jj * 2
      pltpu.async_copy(src(j0 + 1), ibuf.at[1], sem1)
      pltpu.make_async_copy(src(0), ibuf.at[0], sem0).wait()
      f = process_chunk(0, f, base)
      check_flush()

      @pl.when(jj + 1 < nchunks // 2)
      def _():
        pltpu.async_copy(src(j0 + 2), ibuf.at[0], sem0)

      pltpu.make_async_copy(src(0), ibuf.at[1], sem1).wait()
      f = process_chunk(1, f, base)
      check_flush()
      return f

    lax.fori_loop(0, nchunks // 2, pair, zero_vec)
    flush()
    plsc.subcore_barrier()

    w1_vec = w1v[pl.ds(0, L)]
    b1_vec = b1v[pl.ds(0, L)]
    w2_vec = w2v[pl.ds(0, L)]
    b2_vec = b2v[pl.ds(0, L)]
    w1s = [w1_vec[k] for k in range(N_HID)]
    b1s = [b1_vec[k] for k in range(N_HID)]
    w2s = [w2_vec[k] for k in range(N_HID)]
    b2s = b2_vec[0]

    def wbody(j, y):
      off = s * TSLICE + j * WCH
      pltpu.sync_copy(hist.at[pl.ds(off, WCH)], cbuf)
      pltpu.sync_copy(zbuf, hist.at[pl.ds(off, WCH)])

      @plsc.parallel_loop(0, WCH // L, unroll=4)
      def mbody(g):
        cv = cbuf[pl.ds(g * L, L)]
        acc = jnp.full((L,), 0.0, jnp.float32) + b2s
        for k in range(N_HID):
          h = jnp.maximum(cv * w1s[k] + b1s[k], 0.0)
          acc = acc + h * w2s[k]
        obuf[pl.ds(g * L, L)] = acc
      pltpu.sync_copy(obuf, out_hbm.at[pl.ds(base + off, WCH)])
      return y

    lax.fori_loop(0, TSLICE // WCH, wbody, 0)
    plsc.subcore_barrier()
    return x

  lax.fori_loop(0, N_BUCKETS // 2, pass_body, 0)


@jax.jit
def _pooler(idx, w1p, b1p, w2p, b2p):
  mesh = plsc.VectorSubcoreMesh(core_axis_name="c", subcore_axis_name="s")
  f = pl.kernel(
      _sc_body,
      out_type=jax.ShapeDtypeStruct((NB,), jnp.float32),
      mesh=mesh,
      compiler_params=pltpu.CompilerParams(needs_layout_passes=False),
      scratch_types=[
          pltpu.VMEM((2, CH), jnp.int32),
          pltpu.VMEM((SBUF,), jnp.int32),
          pltpu.VMEM((SBUF,), jnp.float32),
          pltpu.VMEM((WCH,), jnp.float32),
          pltpu.VMEM((WCH,), jnp.float32),
          pltpu.VMEM((WCH,), jnp.float32),
          pltpu.VMEM((L,), jnp.float32),
          pltpu.VMEM((L,), jnp.float32),
          pltpu.VMEM((L,), jnp.float32),
          pltpu.VMEM((L,), jnp.float32),
          pltpu.VMEM_SHARED((HSZ,), jnp.float32),
          pltpu.SemaphoreType.DMA,
          pltpu.SemaphoreType.DMA,
      ],
  )
  return f(idx, w1p, b1p, w2p, b2p)


def kernel(cellxgene_ix, weights, n_cells, n_genes, W1, b1, W2, b2):
  del weights, n_cells, n_genes
  idx = cellxgene_ix.astype(jnp.int32)
  assert idx.shape[0] % (L * CH * 2) == 0

  def pad16(a):
    a = a.reshape(-1).astype(jnp.float32)
    return jnp.pad(a, (0, L - a.shape[0]))

  out = _pooler(idx, pad16(W1), pad16(b1), pad16(W2), pad16(b2))
  return out.reshape(N_CELLS, N_GENES)
